# baseline (speedup 1.0000x reference)
import functools

import jax
import jax.numpy as jnp
from jax import lax
from jax.experimental import pallas as pl
from jax.experimental.pallas import tpu as pltpu

_B = 16
_G = 50
_G_PAD = 56
_A = 64 * 64 + 32 * 32 + 16 * 16
_R = 2.5
_BIG_IDX = 1e9
_BIG_COST = 1e30


def _sig(x):
    return 1.0 / (1.0 + jnp.exp(-x))


def _body(p_ref, lab_ref, c_ref, out_ref):
    p6 = p_ref[0]
    lab = lab_ref[0]
    cgrid = c_ref[...]

    gridx = cgrid[0:1, :]
    gridy = cgrid[1:2, :]
    sts = cgrid[2:3, :]
    xc = cgrid[3:4, :]
    yc = cgrid[4:5, :]

    bx = (p6[0:1, :] + gridx) * sts
    by = (p6[1:2, :] + gridy) * sts
    bw = jnp.exp(p6[2:3, :]) * sts
    bh = jnp.exp(p6[3:4, :]) * sts
    obj_l = p6[4:5, :]
    cls_l = p6[5:6, :]

    gx = lab[:, 1:2]
    gy = lab[:, 2:3]
    gw = lab[:, 3:4]
    gh = lab[:, 4:5]

    iota_g = lax.broadcasted_iota(jnp.int32, (_G_PAD, 1), 0).astype(jnp.float32)
    valid = iota_g < float(_G)
    col_iota = lax.broadcasted_iota(jnp.int32, (_G_PAD, _A), 1).astype(jnp.float32)
    row_iota = lax.broadcasted_iota(jnp.int32, (_G_PAD, _A), 0).astype(jnp.float32)

    b_l = xc - (gx - gw / 2)
    b_r = (gx + gw / 2) - xc
    b_t = yc - (gy - gh / 2)
    b_b = (gy + gh / 2) - yc
    mb = jnp.minimum(jnp.minimum(b_l, b_r), jnp.minimum(b_t, b_b))

    c_l = xc - (gx - _R * sts)
    c_r = (gx + _R * sts) - xc
    c_t = yc - (gy - _R * sts)
    c_b = (gy + _R * sts) - yc
    mc = jnp.minimum(jnp.minimum(c_l, c_r), jnp.minimum(c_t, c_b))

    either_v = jnp.maximum(mb, mc)
    both_v = jnp.minimum(mb, mc)
    cand = jnp.max(either_v, axis=0, keepdims=True) > 0.0
    cand_f = jnp.where(cand, 1.0, 0.0)

    ax1 = gx - gw / 2
    ax2 = gx + gw / 2
    ay1 = gy - gh / 2
    ay2 = gy + gh / 2
    px1 = bx - bw / 2
    px2 = bx + bw / 2
    py1 = by - bh / 2
    py2 = by + bh / 2
    wx = jnp.maximum(jnp.minimum(ax2, px2) - jnp.maximum(ax1, px1), 0.0)
    wy = jnp.maximum(jnp.minimum(ay2, py2) - jnp.maximum(ay1, py1), 0.0)
    inter = wx * wy
    area_g = gw * gh
    area_p = bw * bh
    ious = inter / (area_g + area_p - inter + 1e-8)
    ious_c = jnp.where(cand, ious, 0.0)

    p_cls = jnp.sqrt(_sig(cls_l) * _sig(obj_l))
    p_cls = jnp.clip(p_cls, 1e-8, 1.0 - 1e-8)
    cls_cost = -jnp.log(p_cls)
    iou_cost = -jnp.log(ious_c + 1e-8)
    cost = cls_cost + 3.0 * iou_cost + 1e9 * (1.0 - cand_f)
    cost = cost + jnp.where(both_v > 0.0, 0.0, 100000.0)

    w = ious_c
    s = jnp.zeros((_G_PAD, 1), jnp.float32)
    for _ in range(10):
        m = jnp.max(w, axis=1, keepdims=True)
        s = s + jnp.maximum(m, 0.0)
        w = jnp.where(w == m, -1.0, w)
    dyn_k = jnp.maximum(s.astype(jnp.int32), 1)

    work = cost
    match_f = jnp.zeros((_G_PAD, _A), jnp.float32)
    for k in range(10):
        m = jnp.min(work, axis=1, keepdims=True)
        idx = jnp.min(jnp.where(work == m, col_iota, _BIG_IDX),
                      axis=1, keepdims=True)
        ok = (dyn_k > k) & (m < 1e8) & valid
        idx_k = jnp.where(ok, idx, -1.0)
        match_f = match_f + jnp.where(col_iota == idx_k, 1.0, 0.0)
        work = jnp.where(col_iota == idx, _BIG_COST, work)

    amg = jnp.sum(match_f, axis=0, keepdims=True)
    mcol = jnp.min(cost, axis=0, keepdims=True)
    gidx = jnp.min(jnp.where(cost == mcol, row_iota, _BIG_IDX),
                   axis=0, keepdims=True)
    keep_f = jnp.where(row_iota == gidx, 1.0, 0.0)
    m1 = jnp.where(amg > 1.0, keep_f, match_f)

    fg_f = jnp.where(amg > 0.0, 1.0, 0.0)

    rx = jnp.sum(m1 * gx, axis=0, keepdims=True)
    ry = jnp.sum(m1 * gy, axis=0, keepdims=True)
    rw = jnp.sum(m1 * gw, axis=0, keepdims=True)
    rh = jnp.sum(m1 * gh, axis=0, keepdims=True)

    tx1 = rx - rw / 2
    tx2 = rx + rw / 2
    ty1 = ry - rh / 2
    ty2 = ry + rh / 2
    lwx = jnp.maximum(jnp.minimum(px2, tx2) - jnp.maximum(px1, tx1), 0.0)
    lwy = jnp.maximum(jnp.minimum(py2, ty2) - jnp.maximum(py1, ty1), 0.0)
    linter = lwx * lwy
    liou = linter / (bw * bh + rw * rh - linter + 1e-8)
    s_iou = jnp.sum((1.0 - liou * liou) * fg_f)

    t = fg_f
    bce = (jnp.maximum(obj_l, 0.0) - obj_l * t
           + jnp.log1p(jnp.exp(-jnp.abs(obj_l))))
    p_o = _sig(obj_l)
    p_t = t * p_o + (1.0 - t) * (1.0 - p_o)
    af = t * 0.25 + (1.0 - t) * 0.75
    omp = 1.0 - p_t
    s_obj = jnp.sum(bce * af * omp * omp)

    n_fg = jnp.sum(fg_f)

    lane = lax.broadcasted_iota(jnp.int32, (1, 128), 1).astype(jnp.float32)
    e0 = jnp.where(lane == 0.0, 1.0, 0.0)
    e1 = jnp.where(lane == 1.0, 1.0, 0.0)
    e2 = jnp.where(lane == 2.0, 1.0, 0.0)
    vals = s_iou * e0 + s_obj * e1 + n_fg * e2
    row8 = lax.broadcasted_iota(jnp.int32, (8, 128), 0)
    out_ref[...] = jnp.where(row8 == 0, vals, 0.0)[None]


def _reduce_body(acc_ref, out_ref):
    acc = acc_ref[...]
    lane = lax.broadcasted_iota(jnp.int32, (1, 128), 1).astype(jnp.float32)
    e0 = jnp.where(lane == 0.0, 1.0, 0.0)
    e1 = jnp.where(lane == 1.0, 1.0, 0.0)
    e2 = jnp.where(lane == 2.0, 1.0, 0.0)
    tot = jnp.sum(acc, axis=0, keepdims=True)
    a_iou = jnp.sum(tot * e0)
    a_obj = jnp.sum(tot * e1)
    a_nfg = jnp.sum(tot * e2)
    num_fg = jnp.maximum(a_nfg, 1.0)
    l_iou = a_iou / num_fg
    l_obj = a_obj / num_fg
    total = 5.0 * l_iou + l_obj
    out_ref[...] = total * e0 + l_iou * e1 + l_obj * e2


def _anchor_consts():
    rows = []
    for hw, wdim, st in ((4096, 64, 8.0), (1024, 32, 16.0), (256, 16, 32.0)):
        ar = jnp.arange(hw, dtype=jnp.int32)
        xs = (ar % wdim).astype(jnp.float32)
        ys = (ar // wdim).astype(jnp.float32)
        stv = jnp.full((hw,), st, jnp.float32)
        rows.append(jnp.stack([xs, ys, stv]))
    c3 = jnp.concatenate(rows, axis=1)
    xcv = (c3[0] + 0.5) * c3[2]
    ycv = (c3[1] + 0.5) * c3[2]
    return jnp.concatenate(
        [c3, xcv[None], ycv[None], jnp.zeros((3, _A), jnp.float32)], axis=0)


def _run(pred0, pred1, pred2, labels, *, interpret=False):
    bsz = pred0.shape[0]
    p_all = jnp.concatenate(
        [pred0.reshape(bsz, 6, 64 * 64),
         pred1.reshape(bsz, 6, 32 * 32),
         pred2.reshape(bsz, 6, 16 * 16)], axis=2)
    p_all = jnp.pad(p_all, ((0, 0), (0, 2), (0, 0)))

    pad_row = jnp.array([0.0, -1e6, -1e6, 1.0, 1.0], jnp.float32)
    lab = jnp.concatenate(
        [labels, jnp.broadcast_to(pad_row, (bsz, _G_PAD - _G, 5))], axis=1)
    lab = jnp.pad(lab, ((0, 0), (0, 0), (0, 3)))

    consts = _anchor_consts()

    partials = pl.pallas_call(
        _body,
        grid=(bsz,),
        in_specs=[
            pl.BlockSpec((1, 8, _A), lambda b: (b, 0, 0)),
            pl.BlockSpec((1, _G_PAD, 8), lambda b: (b, 0, 0)),
            pl.BlockSpec((8, _A), lambda b: (0, 0)),
        ],
        out_specs=pl.BlockSpec((1, 8, 128), lambda b: (b, 0, 0)),
        out_shape=jax.ShapeDtypeStruct((bsz, 8, 128), jnp.float32),
        compiler_params=pltpu.CompilerParams(
            dimension_semantics=("parallel",)),
        interpret=interpret,
    )(p_all, lab, consts)

    out = pl.pallas_call(
        _reduce_body,
        out_shape=jax.ShapeDtypeStruct((1, 128), jnp.float32),
        interpret=interpret,
    )(partials.reshape(bsz * 8, 128))

    total = out[0, 0]
    loss_iou = out[0, 1]
    loss_obj = out[0, 2]
    return total, loss_iou, loss_obj, jnp.zeros(())


def kernel(pred0, pred1, pred2, labels):
    return _run(pred0, pred1, pred2, labels)


# speedup vs baseline: 1.2020x; 1.2020x over previous
import functools

import jax
import jax.numpy as jnp
from jax import lax
from jax.experimental import pallas as pl
from jax.experimental.pallas import tpu as pltpu

_B = 16
_G = 50
_G_PAD = 56
_A = 64 * 64 + 32 * 32 + 16 * 16
_R = 2.5
_BIG_IDX = 1e9
_BIG_COST = 1e30


def _sig(x):
    return 1.0 / (1.0 + jnp.exp(-x))


def _body(p_ref, lab_ref, labt_ref, c_ref, out_ref):
    p6 = p_ref[0]
    lab = lab_ref[0]
    labt = labt_ref[0]
    cgrid = c_ref[...]

    gridx = cgrid[0:1, :]
    gridy = cgrid[1:2, :]
    sts = cgrid[2:3, :]
    xc = cgrid[3:4, :]
    yc = cgrid[4:5, :]

    bx = (p6[0:1, :] + gridx) * sts
    by = (p6[1:2, :] + gridy) * sts
    bw = jnp.exp(p6[2:3, :]) * sts
    bh = jnp.exp(p6[3:4, :]) * sts
    obj_l = p6[4:5, :]
    cls_l = p6[5:6, :]

    gx = lab[:, 1:2]
    gy = lab[:, 2:3]
    gw = lab[:, 3:4]
    gh = lab[:, 4:5]

    iota_g = lax.broadcasted_iota(jnp.int32, (_G_PAD, 1), 0).astype(jnp.float32)
    valid = iota_g < float(_G)
    col_iota = lax.broadcasted_iota(jnp.int32, (_G_PAD, _A), 1).astype(jnp.float32)
    row_iota = lax.broadcasted_iota(jnp.int32, (_G_PAD, _A), 0).astype(jnp.float32)

    b_l = xc - (gx - gw / 2)
    b_r = (gx + gw / 2) - xc
    b_t = yc - (gy - gh / 2)
    b_b = (gy + gh / 2) - yc
    mb = jnp.minimum(jnp.minimum(b_l, b_r), jnp.minimum(b_t, b_b))

    c_l = xc - (gx - _R * sts)
    c_r = (gx + _R * sts) - xc
    c_t = yc - (gy - _R * sts)
    c_b = (gy + _R * sts) - yc
    mc = jnp.minimum(jnp.minimum(c_l, c_r), jnp.minimum(c_t, c_b))

    either_v = jnp.maximum(mb, mc)
    both_v = jnp.minimum(mb, mc)
    cand = jnp.max(either_v, axis=0, keepdims=True) > 0.0
    cand_f = jnp.where(cand, 1.0, 0.0)

    ax1 = gx - gw / 2
    ax2 = gx + gw / 2
    ay1 = gy - gh / 2
    ay2 = gy + gh / 2
    px1 = bx - bw / 2
    px2 = bx + bw / 2
    py1 = by - bh / 2
    py2 = by + bh / 2
    wx = jnp.maximum(jnp.minimum(ax2, px2) - jnp.maximum(ax1, px1), 0.0)
    wy = jnp.maximum(jnp.minimum(ay2, py2) - jnp.maximum(ay1, py1), 0.0)
    inter = wx * wy
    area_g = gw * gh
    area_p = bw * bh
    ious = inter / (area_g + area_p - inter + 1e-8)
    ious_c = jnp.where(cand, ious, 0.0)

    p_cls = jnp.sqrt(_sig(cls_l) * _sig(obj_l))
    p_cls = jnp.clip(p_cls, 1e-8, 1.0 - 1e-8)
    cls_cost = -jnp.log(p_cls)
    iou_cost = -jnp.log(ious_c + 1e-8)
    cost = cls_cost + 3.0 * iou_cost + 1e9 * (1.0 - cand_f)
    cost = cost + jnp.where(both_v > 0.0, 0.0, 100000.0)

    s = jnp.zeros((_G_PAD, 1), jnp.float32)
    m = jnp.full((_G_PAD, 1), _BIG_COST, jnp.float32)
    for _ in range(10):
        m = jnp.max(jnp.where(ious_c < m, ious_c, -1.0),
                    axis=1, keepdims=True)
        s = s + jnp.maximum(m, 0.0)
    dyn_k = jnp.maximum(s.astype(jnp.int32), 1)

    work = cost
    thr_m = jnp.zeros((_G_PAD, 1), jnp.float32)
    thr_i = jnp.zeros((_G_PAD, 1), jnp.float32)
    dk_f = dyn_k.astype(jnp.float32)
    for k in range(10):
        mk = jnp.min(work, axis=1, keepdims=True)
        idx = jnp.min(jnp.where(work == mk, col_iota, _BIG_IDX),
                      axis=1, keepdims=True)
        work = jnp.where(col_iota == idx, _BIG_COST, work)
        hit = dk_f == float(k + 1)
        thr_m = jnp.where(hit, mk, thr_m)
        thr_i = jnp.where(hit, idx, thr_i)

    match_f = jnp.where(
        ((cost < thr_m) | ((cost == thr_m) & (col_iota <= thr_i)))
        & (cost < 1e8) & valid, 1.0, 0.0)

    amg = jnp.sum(match_f, axis=0, keepdims=True)
    mcol = jnp.min(cost, axis=0, keepdims=True)
    gidx = jnp.min(jnp.where(cost == mcol, row_iota, _BIG_IDX),
                   axis=0, keepdims=True)
    keep_f = jnp.where(row_iota == gidx, 1.0, 0.0)
    m1 = jnp.where(amg > 1.0, keep_f, match_f)

    fg_f = jnp.where(amg > 0.0, 1.0, 0.0)

    boxm = lax.dot_general(labt, m1, (((1,), (0,)), ((), ())),
                           preferred_element_type=jnp.float32)
    rx = boxm[1:2, :]
    ry = boxm[2:3, :]
    rw = boxm[3:4, :]
    rh = boxm[4:5, :]

    tx1 = rx - rw / 2
    tx2 = rx + rw / 2
    ty1 = ry - rh / 2
    ty2 = ry + rh / 2
    lwx = jnp.maximum(jnp.minimum(px2, tx2) - jnp.maximum(px1, tx1), 0.0)
    lwy = jnp.maximum(jnp.minimum(py2, ty2) - jnp.maximum(py1, ty1), 0.0)
    linter = lwx * lwy
    liou = linter / (bw * bh + rw * rh - linter + 1e-8)
    s_iou = jnp.sum((1.0 - liou * liou) * fg_f)

    t = fg_f
    bce = (jnp.maximum(obj_l, 0.0) - obj_l * t
           + jnp.log1p(jnp.exp(-jnp.abs(obj_l))))
    p_o = _sig(obj_l)
    p_t = t * p_o + (1.0 - t) * (1.0 - p_o)
    af = t * 0.25 + (1.0 - t) * 0.75
    omp = 1.0 - p_t
    s_obj = jnp.sum(bce * af * omp * omp)

    n_fg = jnp.sum(fg_f)

    lane = lax.broadcasted_iota(jnp.int32, (1, 128), 1).astype(jnp.float32)
    e0 = jnp.where(lane == 0.0, 1.0, 0.0)
    e1 = jnp.where(lane == 1.0, 1.0, 0.0)
    e2 = jnp.where(lane == 2.0, 1.0, 0.0)
    vals = s_iou * e0 + s_obj * e1 + n_fg * e2
    row8 = lax.broadcasted_iota(jnp.int32, (8, 128), 0)
    out_ref[...] = jnp.where(row8 == 0, vals, 0.0)[None]


def _reduce_body(acc_ref, out_ref):
    acc = acc_ref[...]
    lane = lax.broadcasted_iota(jnp.int32, (1, 128), 1).astype(jnp.float32)
    e0 = jnp.where(lane == 0.0, 1.0, 0.0)
    e1 = jnp.where(lane == 1.0, 1.0, 0.0)
    e2 = jnp.where(lane == 2.0, 1.0, 0.0)
    tot = jnp.sum(acc, axis=0, keepdims=True)
    a_iou = jnp.sum(tot * e0)
    a_obj = jnp.sum(tot * e1)
    a_nfg = jnp.sum(tot * e2)
    num_fg = jnp.maximum(a_nfg, 1.0)
    l_iou = a_iou / num_fg
    l_obj = a_obj / num_fg
    total = 5.0 * l_iou + l_obj
    out_ref[...] = total * e0 + l_iou * e1 + l_obj * e2


def _anchor_consts():
    rows = []
    for hw, wdim, st in ((4096, 64, 8.0), (1024, 32, 16.0), (256, 16, 32.0)):
        ar = jnp.arange(hw, dtype=jnp.int32)
        xs = (ar % wdim).astype(jnp.float32)
        ys = (ar // wdim).astype(jnp.float32)
        stv = jnp.full((hw,), st, jnp.float32)
        rows.append(jnp.stack([xs, ys, stv]))
    c3 = jnp.concatenate(rows, axis=1)
    xcv = (c3[0] + 0.5) * c3[2]
    ycv = (c3[1] + 0.5) * c3[2]
    return jnp.concatenate(
        [c3, xcv[None], ycv[None], jnp.zeros((3, _A), jnp.float32)], axis=0)


def _run(pred0, pred1, pred2, labels, *, interpret=False):
    bsz = pred0.shape[0]
    p_all = jnp.concatenate(
        [pred0.reshape(bsz, 6, 64 * 64),
         pred1.reshape(bsz, 6, 32 * 32),
         pred2.reshape(bsz, 6, 16 * 16)], axis=2)
    p_all = jnp.pad(p_all, ((0, 0), (0, 2), (0, 0)))

    pad_row = jnp.array([0.0, -1e6, -1e6, 1.0, 1.0], jnp.float32)
    lab = jnp.concatenate(
        [labels, jnp.broadcast_to(pad_row, (bsz, _G_PAD - _G, 5))], axis=1)
    lab = jnp.pad(lab, ((0, 0), (0, 0), (0, 3)))
    lab_t = jnp.swapaxes(lab, 1, 2)

    consts = _anchor_consts()

    partials = pl.pallas_call(
        _body,
        grid=(bsz,),
        in_specs=[
            pl.BlockSpec((1, 8, _A), lambda b: (b, 0, 0)),
            pl.BlockSpec((1, _G_PAD, 8), lambda b: (b, 0, 0)),
            pl.BlockSpec((1, 8, _G_PAD), lambda b: (b, 0, 0)),
            pl.BlockSpec((8, _A), lambda b: (0, 0)),
        ],
        out_specs=pl.BlockSpec((1, 8, 128), lambda b: (b, 0, 0)),
        out_shape=jax.ShapeDtypeStruct((bsz, 8, 128), jnp.float32),
        compiler_params=pltpu.CompilerParams(
            dimension_semantics=("parallel",)),
        interpret=interpret,
    )(p_all, lab, lab_t, consts)

    out = pl.pallas_call(
        _reduce_body,
        out_shape=jax.ShapeDtypeStruct((1, 128), jnp.float32),
        interpret=interpret,
    )(partials.reshape(bsz * 8, 128))

    total = out[0, 0]
    loss_iou = out[0, 1]
    loss_obj = out[0, 2]
    return total, loss_iou, loss_obj, jnp.zeros(())


def kernel(pred0, pred1, pred2, labels):
    return _run(pred0, pred1, pred2, labels)
